# baseline (speedup 1.0000x reference)
import functools

import jax
import jax.numpy as jnp
from jax import lax
from jax.experimental import pallas as pl
from jax.experimental.pallas import tpu as pltpu
from jax.experimental.pallas import tpu_sc as plsc


def _make_sc_kernel(n_rows, d, l_n, seg_n, chunk, stage_rows):
    info = plsc.get_sparse_core_info()
    nw = info.num_cores * info.num_subcores
    lanes = info.num_lanes
    assert n_rows % (nw * chunk) == 0
    per_w = n_rows // nw
    n_chunks = per_w // chunk
    assert n_chunks % 4 == 0 and chunk <= 128 and chunk % 8 == 0
    assert stage_rows % 8 == 0 and l_n >= stage_rows
    mesh = plsc.VectorSubcoreMesh(core_axis_name="c", subcore_axis_name="s")

    @functools.partial(
        pl.kernel,
        mesh=mesh,
        out_type=jax.ShapeDtypeStruct((n_rows, d), jnp.float32),
        scratch_types=[
            pltpu.VMEM((n_chunks, chunk), jnp.int32),
            pltpu.VMEM((n_chunks, chunk), jnp.int32),
            pltpu.VMEM_SHARED((seg_n * l_n, d), jnp.float32),
            pltpu.VMEM((stage_rows, d), jnp.float32),
            pltpu.VMEM((seg_n, d), jnp.float32),
            pltpu.VMEM((stage_rows, d), jnp.float32),
            pltpu.VMEM((chunk, d), jnp.float32),
            pltpu.VMEM((chunk, d), jnp.float32),
            pltpu.VMEM((chunk, d), jnp.float32),
            pltpu.VMEM((chunk, d), jnp.float32),
            pltpu.SemaphoreType.DMA,
            pltpu.SemaphoreType.DMA,
            pltpu.SemaphoreType.DMA,
            pltpu.SemaphoreType.DMA,
            pltpu.SemaphoreType.DMA,
            pltpu.SemaphoreType.DMA,
            pltpu.SemaphoreType.DMA,
            pltpu.SemaphoreType.DMA,
            pltpu.SemaphoreType.DMA,
            pltpu.SemaphoreType.DMA,
            pltpu.SemaphoreType.DMA,
            pltpu.SemaphoreType.DMA,
        ],
    )
    def sc_kernel(seq_hbm, seg_hbm, tok_hbm, segtab_hbm, pe_hbm, out_hbm,
                  seqi_v, idx2_v, spe_sh, pe_v, segt_v, stage_v,
                  x0, x1, x2, x3,
                  st0, st1, st2, st3, ss0, ss1, ss2, ss3, so0, so1, so2, so3):
        sid = lax.axis_index("s")
        wid = sid * info.num_cores + lax.axis_index("c")
        base = wid * per_w
        xbufs = (x0, x1, x2, x3)
        sems_t, sems_s, sems_o = (st0, st1, st2, st3), (ss0, ss1, ss2, ss3), (so0, so1, so2, so3)

        lstart = pl.multiple_of(jnp.minimum(sid * stage_rows, l_n - stage_rows), 8)
        pltpu.sync_copy(pe_hbm.at[pl.ds(lstart, stage_rows)], pe_v)
        pltpu.sync_copy(segtab_hbm, segt_v)
        for s in range(seg_n):
            def stage_body(j, carry):
                for c in range(d // lanes):
                    sl = pl.ds(c * lanes, lanes)
                    stage_v[j, sl] = pe_v[j, sl] + segt_v[s, sl]
                return carry

            lax.fori_loop(0, stage_rows, stage_body, 0)
            dst = pl.multiple_of(s * l_n + lstart, 8)
            pltpu.sync_copy(stage_v, spe_sh.at[pl.ds(dst, stage_rows)])

        pltpu.sync_copy(seq_hbm.at[wid], seqi_v)
        pltpu.sync_copy(seg_hbm.at[wid], idx2_v)

        def idx_body(i, carry):
            for j in range(chunk // lanes):
                sl = pl.ds(j * lanes, lanes)
                flat = (base + i * chunk + j * lanes) + lax.iota(jnp.int32, lanes)
                idx2_v[i, sl] = idx2_v[i, sl] * l_n + flat % l_n
            return carry

        lax.fori_loop(0, n_chunks, idx_body, 0)
        plsc.subcore_barrier()

        def tok_dma(g, k):
            return pltpu.make_async_copy(tok_hbm.at[seqi_v.at[g]], xbufs[k], sems_t[k])

        def spe_add_start(g, k):
            return pltpu.async_copy(
                spe_sh.at[idx2_v.at[g]], xbufs[k], sems_s[k], add=True)

        def spe_wait(g, k):
            pltpu.make_async_copy(spe_sh.at[idx2_v.at[g]], xbufs[k], sems_s[k]).wait()

        def out_dma(g, k):
            return pltpu.make_async_copy(
                xbufs[k], out_hbm.at[pl.ds(base + g * chunk, chunk)], sems_o[k])

        tok_dma(0, 0).start()
        tok_dma(1, 1).start()

        def iter_body(t, carry):
            for k in range(4):
                g = 4 * t + k
                tok_dma(g, k).wait()
                spe_add_start(g, k)

                @pl.when(g >= 2)
                def _drain_out():
                    out_dma(g - 2, (k - 2) % 4).wait()

                @pl.when(g + 2 < n_chunks)
                def _prefetch():
                    tok_dma(g + 2, (k + 2) % 4).start()

                spe_wait(g, k)
                out_dma(g, k).start()
            return carry

        lax.fori_loop(0, n_chunks // 4, iter_body, 0)
        out_dma(n_chunks - 2, 2).wait()
        out_dma(n_chunks - 1, 3).wait()

    return sc_kernel


def kernel(sequence, segment, token_table, seg_table, pe):
    b, l_n = sequence.shape
    d = token_table.shape[1]
    seg_n = seg_table.shape[0]
    n_rows = b * l_n
    chunk = 64

    nw = 32
    seq3d = sequence.reshape(nw, n_rows // (nw * chunk), chunk).astype(jnp.int32)
    seg3d = segment.reshape(nw, n_rows // (nw * chunk), chunk).astype(jnp.int32)
    pe2d = pe[0, :l_n]

    sc = _make_sc_kernel(n_rows, d, l_n, seg_n, chunk, stage_rows=16)
    out_flat = sc(seq3d, seg3d, token_table, seg_table, pe2d)
    return out_flat.reshape(b, l_n, d)


# speedup vs baseline: 1.0056x; 1.0056x over previous
import functools

import jax
import jax.numpy as jnp
from jax import lax
from jax.experimental import pallas as pl
from jax.experimental.pallas import tpu as pltpu
from jax.experimental.pallas import tpu_sc as plsc


def _make_sc_kernel(n_rows, d, l_n, seg_n, chunk, stage_rows):
    info = plsc.get_sparse_core_info()
    nw = info.num_cores * info.num_subcores
    lanes = info.num_lanes
    assert n_rows % (nw * chunk) == 0
    per_w = n_rows // nw
    n_chunks = per_w // chunk
    assert n_chunks % 4 == 0 and chunk <= 128 and chunk % 8 == 0
    assert stage_rows % 8 == 0 and l_n >= stage_rows
    mesh = plsc.VectorSubcoreMesh(core_axis_name="c", subcore_axis_name="s")

    @functools.partial(
        pl.kernel,
        mesh=mesh,
        out_type=jax.ShapeDtypeStruct((n_rows, d), jnp.float32),
        scratch_types=[
            pltpu.VMEM((n_chunks, chunk), jnp.int32),
            pltpu.VMEM((n_chunks, chunk), jnp.int32),
            pltpu.VMEM_SHARED((seg_n * l_n, d), jnp.float32),
            pltpu.VMEM((stage_rows, d), jnp.float32),
            pltpu.VMEM((seg_n, d), jnp.float32),
            pltpu.VMEM((stage_rows, d), jnp.float32),
            pltpu.VMEM((chunk, d), jnp.float32),
            pltpu.VMEM((chunk, d), jnp.float32),
            pltpu.VMEM((chunk, d), jnp.float32),
            pltpu.VMEM((chunk, d), jnp.float32),
            pltpu.SemaphoreType.DMA,
            pltpu.SemaphoreType.DMA,
            pltpu.SemaphoreType.DMA,
            pltpu.SemaphoreType.DMA,
            pltpu.SemaphoreType.DMA,
            pltpu.SemaphoreType.DMA,
            pltpu.SemaphoreType.DMA,
            pltpu.SemaphoreType.DMA,
            pltpu.SemaphoreType.DMA,
            pltpu.SemaphoreType.DMA,
            pltpu.SemaphoreType.DMA,
            pltpu.SemaphoreType.DMA,
        ],
    )
    def sc_kernel(seq_hbm, seg_hbm, tok_hbm, segtab_hbm, pe_hbm, out_hbm,
                  seqi_v, idx2_v, spe_sh, pe_v, segt_v, stage_v,
                  x0, x1, x2, x3,
                  st0, st1, st2, st3, ss0, ss1, ss2, ss3, so0, so1, so2, so3):
        sid = lax.axis_index("s")
        wid = sid * info.num_cores + lax.axis_index("c")
        base = wid * per_w
        xbufs = (x0, x1, x2, x3)
        sems_t, sems_s, sems_o = (st0, st1, st2, st3), (ss0, ss1, ss2, ss3), (so0, so1, so2, so3)

        pltpu.sync_copy(seq_hbm.at[wid], seqi_v)
        pltpu.make_async_copy(tok_hbm.at[seqi_v.at[0]], xbufs[0], sems_t[0]).start()
        pltpu.make_async_copy(tok_hbm.at[seqi_v.at[1]], xbufs[1], sems_t[1]).start()

        lstart = pl.multiple_of(jnp.minimum(sid * stage_rows, l_n - stage_rows), 8)
        pltpu.sync_copy(pe_hbm.at[pl.ds(lstart, stage_rows)], pe_v)
        pltpu.sync_copy(segtab_hbm, segt_v)
        for s in range(seg_n):
            def stage_body(j, carry):
                for c in range(d // lanes):
                    sl = pl.ds(c * lanes, lanes)
                    stage_v[j, sl] = pe_v[j, sl] + segt_v[s, sl]
                return carry

            lax.fori_loop(0, stage_rows, stage_body, 0)
            dst = pl.multiple_of(s * l_n + lstart, 8)
            pltpu.sync_copy(stage_v, spe_sh.at[pl.ds(dst, stage_rows)])

        pltpu.sync_copy(seg_hbm.at[wid], idx2_v)

        def idx_body(i, carry):
            for j in range(chunk // lanes):
                sl = pl.ds(j * lanes, lanes)
                flat = (base + i * chunk + j * lanes) + lax.iota(jnp.int32, lanes)
                idx2_v[i, sl] = idx2_v[i, sl] * l_n + flat % l_n
            return carry

        lax.fori_loop(0, n_chunks, idx_body, 0)
        plsc.subcore_barrier()

        def tok_dma(g, k):
            return pltpu.make_async_copy(tok_hbm.at[seqi_v.at[g]], xbufs[k], sems_t[k])

        def spe_add_start(g, k):
            return pltpu.async_copy(
                spe_sh.at[idx2_v.at[g]], xbufs[k], sems_s[k], add=True)

        def spe_wait(g, k):
            pltpu.make_async_copy(spe_sh.at[idx2_v.at[g]], xbufs[k], sems_s[k]).wait()

        def out_dma(g, k):
            return pltpu.make_async_copy(
                xbufs[k], out_hbm.at[pl.ds(base + g * chunk, chunk)], sems_o[k])

        def iter_body(t, carry):
            for k in range(4):
                g = 4 * t + k
                tok_dma(g, k).wait()
                spe_add_start(g, k)

                @pl.when(g >= 2)
                def _drain_out():
                    out_dma(g - 2, (k - 2) % 4).wait()

                @pl.when(g + 2 < n_chunks)
                def _prefetch():
                    tok_dma(g + 2, (k + 2) % 4).start()

                spe_wait(g, k)
                out_dma(g, k).start()
            return carry

        lax.fori_loop(0, n_chunks // 4, iter_body, 0)
        out_dma(n_chunks - 2, 2).wait()
        out_dma(n_chunks - 1, 3).wait()

    return sc_kernel


def kernel(sequence, segment, token_table, seg_table, pe):
    b, l_n = sequence.shape
    d = token_table.shape[1]
    seg_n = seg_table.shape[0]
    n_rows = b * l_n
    chunk = 64

    nw = 32
    seq3d = sequence.reshape(nw, n_rows // (nw * chunk), chunk).astype(jnp.int32)
    seg3d = segment.reshape(nw, n_rows // (nw * chunk), chunk).astype(jnp.int32)
    pe2d = pe[0, :l_n]

    sc = _make_sc_kernel(n_rows, d, l_n, seg_n, chunk, stage_rows=16)
    out_flat = sc(seq3d, seg3d, token_table, seg_table, pe2d)
    return out_flat.reshape(b, l_n, d)


# speedup vs baseline: 1.0540x; 1.0482x over previous
import functools

import jax
import jax.numpy as jnp
from jax import lax
from jax.experimental import pallas as pl
from jax.experimental.pallas import tpu as pltpu
from jax.experimental.pallas import tpu_sc as plsc


def _make_sc_kernel(n_rows, d, l_n, seg_n, chunk, stage_rows):
    info = plsc.get_sparse_core_info()
    nw = info.num_cores * info.num_subcores
    lanes = info.num_lanes
    assert n_rows % (nw * chunk) == 0
    per_w = n_rows // nw
    n_chunks = per_w // chunk
    assert n_chunks % 4 == 0 and chunk <= 128 and chunk % 8 == 0
    assert stage_rows % 8 == 0 and l_n >= stage_rows
    mesh = plsc.VectorSubcoreMesh(core_axis_name="c", subcore_axis_name="s")

    @functools.partial(
        pl.kernel,
        mesh=mesh,
        out_type=jax.ShapeDtypeStruct((n_rows, d), jnp.float32),
        scratch_types=[
            pltpu.VMEM((n_chunks, chunk), jnp.int32),
            pltpu.VMEM((n_chunks, chunk), jnp.int32),
            pltpu.VMEM_SHARED((seg_n * l_n, d), jnp.float32),
            pltpu.VMEM((stage_rows, d), jnp.float32),
            pltpu.VMEM((seg_n, d), jnp.float32),
            pltpu.VMEM((stage_rows, d), jnp.float32),
            pltpu.VMEM((chunk, d), jnp.float32),
            pltpu.VMEM((chunk, d), jnp.float32),
            pltpu.VMEM((chunk, d), jnp.float32),
            pltpu.VMEM((chunk, d), jnp.float32),
            pltpu.SemaphoreType.DMA,
            pltpu.SemaphoreType.DMA,
            pltpu.SemaphoreType.DMA,
            pltpu.SemaphoreType.DMA,
            pltpu.SemaphoreType.DMA,
            pltpu.SemaphoreType.DMA,
            pltpu.SemaphoreType.DMA,
            pltpu.SemaphoreType.DMA,
            pltpu.SemaphoreType.DMA,
            pltpu.SemaphoreType.DMA,
            pltpu.SemaphoreType.DMA,
            pltpu.SemaphoreType.DMA,
        ],
    )
    def sc_kernel(seq_hbm, seg_hbm, tok_hbm, segtab_hbm, pe_hbm, out_hbm,
                  seqi_v, idx2_v, spe_sh, pe_v, segt_v, stage_v,
                  x0, x1, x2, x3,
                  st0, st1, st2, st3, ss0, ss1, ss2, ss3, so0, so1, so2, so3):
        sid = lax.axis_index("s")
        wid = sid * info.num_cores + lax.axis_index("c")
        base = wid * per_w
        xbufs = (x0, x1, x2, x3)
        sems_t, sems_s, sems_o = (st0, st1, st2, st3), (ss0, ss1, ss2, ss3), (so0, so1, so2, so3)

        pltpu.sync_copy(seq_hbm.at[wid], seqi_v)
        pltpu.make_async_copy(tok_hbm.at[seqi_v.at[0]], xbufs[0], sems_t[0]).start()
        pltpu.make_async_copy(tok_hbm.at[seqi_v.at[1]], xbufs[1], sems_t[1]).start()

        lstart = pl.multiple_of(jnp.minimum(sid * stage_rows, l_n - stage_rows), 8)
        pltpu.sync_copy(pe_hbm.at[pl.ds(lstart, stage_rows)], pe_v)
        pltpu.sync_copy(segtab_hbm, segt_v)
        for s in range(seg_n):
            def stage_body(j, carry):
                for c in range(d // lanes):
                    sl = pl.ds(c * lanes, lanes)
                    stage_v[j, sl] = pe_v[j, sl] + segt_v[s, sl]
                return carry

            lax.fori_loop(0, stage_rows, stage_body, 0)
            dst = pl.multiple_of(s * l_n + lstart, 8)
            pltpu.sync_copy(stage_v, spe_sh.at[pl.ds(dst, stage_rows)])

        pltpu.sync_copy(seg_hbm.at[wid], idx2_v)

        def idx_body(i, carry):
            for j in range(chunk // lanes):
                sl = pl.ds(j * lanes, lanes)
                flat = (base + i * chunk + j * lanes) + lax.iota(jnp.int32, lanes)
                idx2_v[i, sl] = idx2_v[i, sl] * l_n + flat % l_n
            return carry

        lax.fori_loop(0, n_chunks, idx_body, 0)
        plsc.subcore_barrier()

        def tok_dma(g, k):
            return pltpu.make_async_copy(tok_hbm.at[seqi_v.at[g]], xbufs[k], sems_t[k])

        def spe_add_start(g, k):
            return pltpu.async_copy(
                spe_sh.at[idx2_v.at[g]], xbufs[k], sems_s[k], add=True)

        def spe_wait(g, k):
            pltpu.make_async_copy(spe_sh.at[idx2_v.at[g]], xbufs[k], sems_s[k]).wait()

        def out_dma(g, k):
            return pltpu.make_async_copy(
                xbufs[k], out_hbm.at[pl.ds(base + g * chunk, chunk)], sems_o[k])

        def iter_body(t, carry):
            for k in range(4):
                g = 4 * t + k
                tok_dma(g, k).wait()
                spe_add_start(g, k)

                @pl.when(g >= 2)
                def _drain_out():
                    out_dma(g - 2, (k - 2) % 4).wait()

                @pl.when(g + 2 < n_chunks)
                def _prefetch():
                    tok_dma(g + 2, (k + 2) % 4).start()

                spe_wait(g, k)
                out_dma(g, k).start()
            return carry

        lax.fori_loop(0, n_chunks // 4, iter_body, 0)
        out_dma(n_chunks - 2, 2).wait()
        out_dma(n_chunks - 1, 3).wait()

    return sc_kernel


def kernel(sequence, segment, token_table, seg_table, pe):
    b, l_n = sequence.shape
    d = token_table.shape[1]
    seg_n = seg_table.shape[0]
    n_rows = b * l_n
    chunk = 80

    nw = 32
    seq3d = sequence.reshape(nw, n_rows // (nw * chunk), chunk).astype(jnp.int32)
    seg3d = segment.reshape(nw, n_rows // (nw * chunk), chunk).astype(jnp.int32)
    pe2d = pe[0, :l_n]

    sc = _make_sc_kernel(n_rows, d, l_n, seg_n, chunk, stage_rows=16)
    out_flat = sc(seq3d, seg3d, token_table, seg_table, pe2d)
    return out_flat.reshape(b, l_n, d)


# speedup vs baseline: 1.0717x; 1.0168x over previous
import functools

import jax
import jax.numpy as jnp
from jax import lax
from jax.experimental import pallas as pl
from jax.experimental.pallas import tpu as pltpu
from jax.experimental.pallas import tpu_sc as plsc


def _make_sc_kernel(n_rows, d, l_n, seg_n, chunk, stage_rows):
    info = plsc.get_sparse_core_info()
    nw = info.num_cores * info.num_subcores
    lanes = info.num_lanes
    assert n_rows % (nw * chunk) == 0
    per_w = n_rows // nw
    n_chunks = per_w // chunk
    assert (n_chunks - 2) % 4 == 0 and chunk <= 128 and chunk % 8 == 0
    assert stage_rows % 8 == 0 and l_n >= stage_rows
    mesh = plsc.VectorSubcoreMesh(core_axis_name="c", subcore_axis_name="s")

    @functools.partial(
        pl.kernel,
        mesh=mesh,
        out_type=jax.ShapeDtypeStruct((n_rows, d), jnp.float32),
        scratch_types=[
            pltpu.VMEM((n_chunks, chunk), jnp.int32),
            pltpu.VMEM((n_chunks, chunk), jnp.int32),
            pltpu.VMEM_SHARED((seg_n * l_n, d), jnp.float32),
            pltpu.VMEM((stage_rows, d), jnp.float32),
            pltpu.VMEM((seg_n, d), jnp.float32),
            pltpu.VMEM((stage_rows, d), jnp.float32),
            pltpu.VMEM((chunk, d), jnp.float32),
            pltpu.VMEM((chunk, d), jnp.float32),
            pltpu.VMEM((chunk, d), jnp.float32),
            pltpu.VMEM((chunk, d), jnp.float32),
            pltpu.SemaphoreType.DMA,
            pltpu.SemaphoreType.DMA,
            pltpu.SemaphoreType.DMA,
            pltpu.SemaphoreType.DMA,
            pltpu.SemaphoreType.DMA,
            pltpu.SemaphoreType.DMA,
            pltpu.SemaphoreType.DMA,
            pltpu.SemaphoreType.DMA,
            pltpu.SemaphoreType.DMA,
            pltpu.SemaphoreType.DMA,
            pltpu.SemaphoreType.DMA,
            pltpu.SemaphoreType.DMA,
        ],
    )
    def sc_kernel(seq_hbm, seg_hbm, tok_hbm, segtab_hbm, pe_hbm, out_hbm,
                  seqi_v, idx2_v, spe_sh, pe_v, segt_v, stage_v,
                  x0, x1, x2, x3,
                  st0, st1, st2, st3, ss0, ss1, ss2, ss3, so0, so1, so2, so3):
        sid = lax.axis_index("s")
        wid = sid * info.num_cores + lax.axis_index("c")
        base = wid * per_w
        xbufs = (x0, x1, x2, x3)
        sems_t, sems_s, sems_o = (st0, st1, st2, st3), (ss0, ss1, ss2, ss3), (so0, so1, so2, so3)

        pltpu.sync_copy(seq_hbm.at[wid], seqi_v)
        pltpu.make_async_copy(tok_hbm.at[seqi_v.at[0]], xbufs[0], sems_t[0]).start()
        pltpu.make_async_copy(tok_hbm.at[seqi_v.at[1]], xbufs[1], sems_t[1]).start()

        lstart = pl.multiple_of(jnp.minimum(sid * stage_rows, l_n - stage_rows), 8)
        pltpu.sync_copy(pe_hbm.at[pl.ds(lstart, stage_rows)], pe_v)
        pltpu.sync_copy(segtab_hbm, segt_v)
        for s in range(seg_n):
            def stage_body(j, carry):
                for c in range(d // lanes):
                    sl = pl.ds(c * lanes, lanes)
                    stage_v[j, sl] = pe_v[j, sl] + segt_v[s, sl]
                return carry

            lax.fori_loop(0, stage_rows, stage_body, 0)
            dst = pl.multiple_of(s * l_n + lstart, 8)
            pltpu.sync_copy(stage_v, spe_sh.at[pl.ds(dst, stage_rows)])

        pltpu.sync_copy(seg_hbm.at[wid], idx2_v)

        def idx_body(i, carry):
            for j in range(chunk // lanes):
                sl = pl.ds(j * lanes, lanes)
                flat = (base + i * chunk + j * lanes) + lax.iota(jnp.int32, lanes)
                idx2_v[i, sl] = idx2_v[i, sl] * l_n + flat % l_n
            return carry

        lax.fori_loop(0, n_chunks, idx_body, 0)
        plsc.subcore_barrier()

        def tok_dma(g, k):
            return pltpu.make_async_copy(tok_hbm.at[seqi_v.at[g]], xbufs[k], sems_t[k])

        def spe_add_start(g, k):
            return pltpu.async_copy(
                spe_sh.at[idx2_v.at[g]], xbufs[k], sems_s[k], add=True)

        def spe_wait(g, k):
            pltpu.make_async_copy(spe_sh.at[idx2_v.at[g]], xbufs[k], sems_s[k]).wait()

        def out_dma(g, k):
            return pltpu.make_async_copy(
                xbufs[k], out_hbm.at[pl.ds(base + g * chunk, chunk)], sems_o[k])

        def iter_body(t, carry):
            for k in range(4):
                g = 4 * t + k
                tok_dma(g, k).wait()
                spe_add_start(g, k)

                @pl.when(g >= 2)
                def _drain_out():
                    out_dma(g - 2, (k - 2) % 4).wait()

                @pl.when(g + 2 < n_chunks)
                def _prefetch():
                    tok_dma(g + 2, (k + 2) % 4).start()

                spe_wait(g, k)
                out_dma(g, k).start()
            return carry

        lax.fori_loop(0, (n_chunks - 2) // 4, iter_body, 0)
        for g in (n_chunks - 2, n_chunks - 1):
            k = g % 4
            tok_dma(g, k).wait()
            spe_add_start(g, k)
            out_dma(g - 2, (k - 2) % 4).wait()
            spe_wait(g, k)
            out_dma(g, k).start()
        out_dma(n_chunks - 2, (n_chunks - 2) % 4).wait()
        out_dma(n_chunks - 1, (n_chunks - 1) % 4).wait()

    return sc_kernel


def kernel(sequence, segment, token_table, seg_table, pe):
    b, l_n = sequence.shape
    d = token_table.shape[1]
    seg_n = seg_table.shape[0]
    n_rows = b * l_n
    chunk = 128

    nw = 32
    seq3d = sequence.reshape(nw, n_rows // (nw * chunk), chunk).astype(jnp.int32)
    seg3d = segment.reshape(nw, n_rows // (nw * chunk), chunk).astype(jnp.int32)
    pe2d = pe[0, :l_n]

    sc = _make_sc_kernel(n_rows, d, l_n, seg_n, chunk, stage_rows=16)
    out_flat = sc(seq3d, seg3d, token_table, seg_table, pe2d)
    return out_flat.reshape(b, l_n, d)
